# baseline (speedup 1.0000x reference)
import functools

import jax
import jax.numpy as jnp
from jax import lax
from jax.experimental import pallas as pl
from jax.experimental.pallas import tpu as pltpu
from jax.experimental.pallas import tpu_sc as plsc

_S = 64.0
_M = 0.4

_CH = 16000
_UNROLL = 8


@functools.cache
def _sc_cosface(rows, vocab):
    info = plsc.get_sparse_core_info()
    nc, lanes = info.num_cores, info.num_lanes
    nw = nc * info.num_subcores
    per_w = rows // nw
    n_per_w = per_w * vocab
    nch = n_per_w // _CH
    assert n_per_w % _CH == 0 and _CH % (lanes * _UNROLL) == 0 and nch >= 4
    mesh = plsc.VectorSubcoreMesh(core_axis_name="c", subcore_axis_name="s")

    @functools.partial(
        pl.kernel,
        out_type=jax.ShapeDtypeStruct((rows * vocab,), jnp.float32),
        mesh=mesh,
        scratch_types=[
            pltpu.VMEM((_CH,), jnp.float32),
            pltpu.VMEM((_CH,), jnp.float32),
            pltpu.VMEM((_CH,), jnp.float32),
            pltpu.VMEM((_CH,), jnp.float32),
            pltpu.VMEM((per_w,), jnp.int32),
            pltpu.VMEM((per_w,), jnp.int32),
            pltpu.VMEM((per_w,), jnp.float32),
            pltpu.SemaphoreType.DMA,
            pltpu.SemaphoreType.DMA,
            pltpu.SemaphoreType.DMA,
            pltpu.SemaphoreType.DMA,
            pltpu.SemaphoreType.DMA,
        ],
    )
    def cosface(x_hbm, lab_hbm, o_hbm, ib0, ib1, ob0, ob1, lab_v, idx_v,
                val_v, is0, is1, os0, os1, fsem):
        wid = lax.axis_index("s") * nc + lax.axis_index("c")
        elem0 = wid * n_per_w
        ibufs, obufs = (ib0, ib1), (ob0, ob1)
        isems, osems = (is0, is1), (os0, os1)

        def chunk_off(idx):
            return elem0 + idx * _CH

        def start_in(idx, b):
            pltpu.make_async_copy(
                x_hbm.at[pl.ds(chunk_off(idx), _CH)], ibufs[b], isems[b]
            ).start()

        def start_out(idx, b):
            pltpu.make_async_copy(
                obufs[b], o_hbm.at[pl.ds(chunk_off(idx), _CH)], osems[b]
            ).start()

        def wait_in(b):
            pltpu.make_async_copy(
                x_hbm.at[pl.ds(elem0, _CH)], ibufs[b], isems[b]
            ).wait()

        def wait_out(b):
            pltpu.make_async_copy(
                obufs[b], o_hbm.at[pl.ds(elem0, _CH)], osems[b]
            ).wait()

        def compute(b):
            ib, ob = ibufs[b], obufs[b]

            def body(j, carry):
                base = j * (lanes * _UNROLL)
                for u in range(_UNROLL):
                    off = base + u * lanes
                    ob[pl.ds(off, lanes)] = ib[pl.ds(off, lanes)] * jnp.float32(_S)
                return carry

            lax.fori_loop(0, _CH // (lanes * _UNROLL), body, 0)

        start_in(0, 0)
        start_in(1, 1)
        for b in range(2):
            wait_in(b)
            compute(b)
            start_out(b, b)
            start_in(2 + b, b)

        def group(g, carry):
            for b in range(2):
                idx = 2 * g + b
                wait_in(b)
                wait_out(b)
                compute(b)
                start_out(idx, b)
                start_in(idx + 2, b)
            return carry

        lax.fori_loop(1, nch // 2 - 1, group, 0)

        for b in range(2):
            idx = nch - 2 + b
            wait_in(b)
            wait_out(b)
            compute(b)
            start_out(idx, b)
        for b in range(2):
            wait_out(b)

        row0 = wid * per_w
        pltpu.sync_copy(lab_hbm.at[pl.ds(row0, per_w)], lab_v)
        for k in range(per_w // lanes):
            lab = lab_v[pl.ds(k * lanes, lanes)]
            row = row0 + k * lanes + lax.iota(jnp.int32, lanes)
            idx_v[pl.ds(k * lanes, lanes)] = row * vocab + jnp.maximum(lab, 0)
        pltpu.async_copy(o_hbm.at[idx_v], val_v, fsem).wait()
        for k in range(per_w // lanes):
            lab = lab_v[pl.ds(k * lanes, lanes)]
            val = val_v[pl.ds(k * lanes, lanes)]
            margin = jnp.where(lab >= 0, jnp.float32(_S * _M), jnp.float32(0.0))
            val_v[pl.ds(k * lanes, lanes)] = val - margin
        pltpu.async_copy(val_v, o_hbm.at[idx_v], fsem).wait()

    return cosface


def kernel(logits, labels):
    rows, vocab = logits.shape
    out = _sc_cosface(rows, vocab)(logits.reshape(-1), labels.astype(jnp.int32))
    return out.reshape(rows, vocab)


# speedup vs baseline: 1.0007x; 1.0007x over previous
import functools

import jax
import jax.numpy as jnp
from jax import lax
from jax.experimental import pallas as pl
from jax.experimental.pallas import tpu as pltpu
from jax.experimental.pallas import tpu_sc as plsc

_S = 64.0
_M = 0.4

_CH = 16000
_UNROLL = 8


@functools.cache
def _sc_cosface(rows, vocab):
    info = plsc.get_sparse_core_info()
    nc, lanes = info.num_cores, info.num_lanes
    nw = nc * info.num_subcores
    per_w = rows // nw
    n_per_w = per_w * vocab
    nch = n_per_w // _CH
    assert n_per_w % _CH == 0 and _CH % (lanes * _UNROLL) == 0 and nch >= 4
    mesh = plsc.VectorSubcoreMesh(core_axis_name="c", subcore_axis_name="s")

    @functools.partial(
        pl.kernel,
        out_type=jax.ShapeDtypeStruct((rows * vocab,), jnp.float32),
        mesh=mesh,
        scratch_types=[
            pltpu.VMEM((_CH,), jnp.float32),
            pltpu.VMEM((_CH,), jnp.float32),
            pltpu.VMEM((_CH,), jnp.float32),
            pltpu.VMEM((_CH,), jnp.float32),
            pltpu.VMEM((per_w,), jnp.int32),
            pltpu.VMEM((per_w,), jnp.int32),
            pltpu.VMEM((per_w,), jnp.float32),
            pltpu.SemaphoreType.DMA,
            pltpu.SemaphoreType.DMA,
            pltpu.SemaphoreType.DMA,
            pltpu.SemaphoreType.DMA,
            pltpu.SemaphoreType.DMA,
        ],
    )
    def cosface(x_hbm, lab_hbm, o_hbm, ib0, ib1, ob0, ob1, lab_v, idx_v,
                val_v, is0, is1, os0, os1, fsem):
        wid = lax.axis_index("s") * nc + lax.axis_index("c")
        elem0 = wid * n_per_w
        ibufs, obufs = (ib0, ib1), (ob0, ob1)
        isems, osems = (is0, is1), (os0, os1)

        def chunk_off(idx):
            return elem0 + idx * _CH

        def start_in(idx, b):
            pltpu.make_async_copy(
                x_hbm.at[pl.ds(chunk_off(idx), _CH)], ibufs[b], isems[b]
            ).start()

        def start_out(idx, b):
            pltpu.make_async_copy(
                obufs[b], o_hbm.at[pl.ds(chunk_off(idx), _CH)], osems[b]
            ).start()

        def wait_in(b):
            pltpu.make_async_copy(
                x_hbm.at[pl.ds(elem0, _CH)], ibufs[b], isems[b]
            ).wait()

        def wait_out(b):
            pltpu.make_async_copy(
                obufs[b], o_hbm.at[pl.ds(elem0, _CH)], osems[b]
            ).wait()

        def compute(b):
            ib, ob = ibufs[b], obufs[b]

            @plsc.parallel_loop(0, _CH, step=lanes, unroll=_UNROLL)
            def body(off):
                ob[pl.ds(off, lanes)] = ib[pl.ds(off, lanes)] * jnp.float32(_S)

        start_in(0, 0)
        start_in(1, 1)
        for b in range(2):
            wait_in(b)
            compute(b)
            start_out(b, b)
            start_in(2 + b, b)

        def group(g, carry):
            for b in range(2):
                idx = 2 * g + b
                wait_in(b)
                wait_out(b)
                compute(b)
                start_out(idx, b)
                start_in(idx + 2, b)
            return carry

        lax.fori_loop(1, nch // 2 - 1, group, 0)

        for b in range(2):
            idx = nch - 2 + b
            wait_in(b)
            wait_out(b)
            compute(b)
            start_out(idx, b)
        for b in range(2):
            wait_out(b)

        row0 = wid * per_w
        pltpu.sync_copy(lab_hbm.at[pl.ds(row0, per_w)], lab_v)
        for k in range(per_w // lanes):
            lab = lab_v[pl.ds(k * lanes, lanes)]
            row = row0 + k * lanes + lax.iota(jnp.int32, lanes)
            idx_v[pl.ds(k * lanes, lanes)] = row * vocab + jnp.maximum(lab, 0)
        pltpu.async_copy(o_hbm.at[idx_v], val_v, fsem).wait()
        for k in range(per_w // lanes):
            lab = lab_v[pl.ds(k * lanes, lanes)]
            val = val_v[pl.ds(k * lanes, lanes)]
            margin = jnp.where(lab >= 0, jnp.float32(_S * _M), jnp.float32(0.0))
            val_v[pl.ds(k * lanes, lanes)] = val - margin
        pltpu.async_copy(val_v, o_hbm.at[idx_v], fsem).wait()

    return cosface


def kernel(logits, labels):
    rows, vocab = logits.shape
    out = _sc_cosface(rows, vocab)(logits.reshape(-1), labels.astype(jnp.int32))
    return out.reshape(rows, vocab)
